# Initial kernel scaffold; baseline (speedup 1.0000x reference)
import jax
import jax.numpy as jnp
from jax.experimental import pallas as pl


def kernel(x, W, a, FC):
    raise NotImplementedError("write your pallas kernel here")


# speedup vs baseline: 2.7460x; 2.7460x over previous
import jax
import jax.numpy as jnp
from jax.experimental import pallas as pl

NFEAT = 128
NHID = 32
NCLASS = 16
NHEADS = 8
ALPHA = 0.2
B = 4
N = 1024


def _gat_kernel(x_ref, w_ref, a_ref, fc_ref, out_ref):
    h_idx = pl.program_id(1)

    x = x_ref[0]
    w = w_ref[0]
    av = a_ref[0, 0]
    hm = jnp.dot(x, w, preferred_element_type=jnp.float32)

    a12 = jnp.stack([av[:NHID], av[NHID:]], axis=1)
    f = jnp.dot(hm, a12, preferred_element_type=jnp.float32)
    f1 = f[:, 0:1]
    f2 = f[:, 1:2]

    mx = jnp.max(f2)
    e = f1 + f2.T
    e = jnp.where(e > 0, e, ALPHA * e)
    m = f1 + mx
    m = jnp.where(m > 0, m, ALPHA * m)
    p = jnp.exp(e - m)
    s = jnp.sum(p, axis=1, keepdims=True)
    num = jnp.dot(p, hm, preferred_element_type=jnp.float32)
    hp = num / s
    hp = jnp.where(hp > 0, hp, jnp.exp(jnp.minimum(hp, 0.0)) - 1.0)
    pooled = jnp.max(hp, axis=0, keepdims=True)

    contrib = jnp.dot(pooled, fc_ref[0], preferred_element_type=jnp.float32)

    @pl.when(h_idx == 0)
    def _():
        out_ref[0] = contrib

    @pl.when(h_idx != 0)
    def _():
        out_ref[0] += contrib


def kernel(x, W, a, FC):
    a2d = a.reshape(NHEADS, 1, 2 * NHID)
    fc3d = FC.reshape(NHEADS, NHID, NCLASS)
    out = pl.pallas_call(
        _gat_kernel,
        grid=(B, NHEADS),
        in_specs=[
            pl.BlockSpec((1, N, NFEAT), lambda b, h: (b, 0, 0)),
            pl.BlockSpec((1, NFEAT, NHID), lambda b, h: (h, 0, 0)),
            pl.BlockSpec((1, 1, 2 * NHID), lambda b, h: (h, 0, 0)),
            pl.BlockSpec((1, NHID, NCLASS), lambda b, h: (h, 0, 0)),
        ],
        out_specs=pl.BlockSpec((1, 1, NCLASS), lambda b, h: (b, 0, 0)),
        out_shape=jax.ShapeDtypeStruct((B, 1, NCLASS), jnp.float32),
    )(x, W, a2d, fc3d)
    return out.reshape(B, NCLASS)


# speedup vs baseline: 2.8768x; 1.0476x over previous
import jax
import jax.numpy as jnp
from jax.experimental import pallas as pl
from jax.experimental.pallas import tpu as pltpu

NFEAT = 128
NHID = 32
NCLASS = 16
NHEADS = 8
ALPHA = 0.2
B = 4
N = 1024


def _gat_kernel(x_ref, w_ref, a_ref, fc_ref, out_ref):
    h_idx = pl.program_id(1)

    x = x_ref[0]
    w = w_ref[0]
    av = a_ref[0, 0]
    hm = jnp.dot(x, w, preferred_element_type=jnp.float32)

    a12 = jnp.stack([av[:NHID], av[NHID:]], axis=1)
    f = jnp.dot(hm, a12, preferred_element_type=jnp.float32)
    f1 = f[:, 0:1]
    f2 = f[:, 1:2]

    mx = jnp.max(f2)
    m = f1 + mx
    m = jnp.where(m > 0, m, ALPHA * m)
    g1 = f1 - m
    g1a = ALPHA * f1 - m
    g2 = f2.T
    g2a = ALPHA * g2
    p = jnp.exp(jnp.maximum(g1 + g2, g1a + g2a))
    hm_ext = jnp.concatenate([hm, jnp.ones((N, 1), jnp.float32)], axis=1)
    num = jnp.dot(p, hm_ext, preferred_element_type=jnp.float32)
    hp = num[:, :NHID] / num[:, NHID:]
    hp = jnp.where(hp > 0, hp, jnp.exp(jnp.minimum(hp, 0.0)) - 1.0)
    pooled = jnp.max(hp, axis=0, keepdims=True)

    contrib = jnp.dot(pooled, fc_ref[0], preferred_element_type=jnp.float32)

    @pl.when(h_idx == 0)
    def _():
        out_ref[0] = contrib

    @pl.when(h_idx != 0)
    def _():
        out_ref[0] += contrib


def kernel(x, W, a, FC):
    a2d = a.reshape(NHEADS, 1, 2 * NHID)
    fc3d = FC.reshape(NHEADS, NHID, NCLASS)
    out = pl.pallas_call(
        _gat_kernel,
        grid=(B, NHEADS),
        in_specs=[
            pl.BlockSpec((1, N, NFEAT), lambda b, h: (b, 0, 0)),
            pl.BlockSpec((1, NFEAT, NHID), lambda b, h: (h, 0, 0)),
            pl.BlockSpec((1, 1, 2 * NHID), lambda b, h: (h, 0, 0)),
            pl.BlockSpec((1, NHID, NCLASS), lambda b, h: (h, 0, 0)),
        ],
        out_specs=pl.BlockSpec((1, 1, NCLASS), lambda b, h: (b, 0, 0)),
        out_shape=jax.ShapeDtypeStruct((B, 1, NCLASS), jnp.float32),
        compiler_params=pltpu.CompilerParams(
            dimension_semantics=("parallel", "arbitrary"),
        ),
    )(x, W, a2d, fc3d)
    return out.reshape(B, NCLASS)
